# baseline (speedup 1.0000x reference)
import functools

import jax
import jax.numpy as jnp
from jax import lax
from jax.experimental import pallas as pl
from jax.experimental.pallas import tpu as pltpu
from jax.experimental.pallas import tpu_sc as plsc

_LANES = 16
_NUM_WORKERS = 32
_ROWS = 8
_T_CHUNK = 2048
_NIN = 4
_NOUT = 2


def _make_resample(b: int, c: int, t: int):
    o_chunk = _T_CHUNK // 2
    strips_total = (b * c) // _ROWS
    strips_per_w = strips_total // _NUM_WORKERS
    chunks_per_strip = t // _T_CHUNK
    chunks = strips_per_w * chunks_per_strip
    assert strips_per_w * _NUM_WORKERS == strips_total
    assert chunks_per_strip * _T_CHUNK == t and chunks % (_NIN * _NOUT) == 0
    strips_per_batch = c // _ROWS

    mesh = plsc.VectorSubcoreMesh(core_axis_name="c", subcore_axis_name="s")

    @functools.partial(
        pl.kernel,
        mesh=mesh,
        compiler_params=pltpu.CompilerParams(needs_layout_passes=False),
        out_type=jax.ShapeDtypeStruct((b, c, t // 2), jnp.float32),
        scratch_types=(
            [pltpu.VMEM((_ROWS, _T_CHUNK), jnp.float32)] * _NIN
            + [pltpu.VMEM((_ROWS, o_chunk), jnp.float32)] * _NOUT
            + [pltpu.SemaphoreType.DMA] * (_NIN + _NOUT)
        ),
    )
    def resample(x_hbm, out_hbm, *bufs):
        in_bufs = bufs[:_NIN]
        out_bufs = bufs[_NIN:_NIN + _NOUT]
        in_sems = bufs[_NIN + _NOUT:_NIN + _NOUT + _NIN]
        out_sems = bufs[_NIN + _NOUT + _NIN:]
        wid = lax.axis_index("s") * 2 + lax.axis_index("c")
        strip0 = wid * strips_per_w
        even = lax.iota(jnp.int32, _LANES) * 2

        def in_copy(i, sl):
            strip = strip0 + i // chunks_per_strip
            t0 = (i % chunks_per_strip) * _T_CHUNK
            src = x_hbm.at[strip // strips_per_batch,
                           pl.ds((strip % strips_per_batch) * _ROWS, _ROWS),
                           pl.ds(t0, _T_CHUNK)]
            return pltpu.make_async_copy(src, in_bufs[sl], in_sems[sl])

        def out_copy(i, sl):
            strip = strip0 + i // chunks_per_strip
            t0 = (i % chunks_per_strip) * o_chunk
            dst = out_hbm.at[strip // strips_per_batch,
                             pl.ds((strip % strips_per_batch) * _ROWS, _ROWS),
                             pl.ds(t0, o_chunk)]
            return pltpu.make_async_copy(out_bufs[sl], dst, out_sems[sl])

        for i in range(_NIN - 1):
            in_copy(i, i).start()

        def outer(g, carry):
            for u in range(_NIN):
                i = g + u
                si = u
                so = u % _NOUT
                in_copy(i, si).wait()

                @pl.when(i + _NIN - 1 < chunks)
                def _():
                    in_copy(i + _NIN - 1, (si + _NIN - 1) % _NIN).start()

                @pl.when(i >= _NOUT)
                def _():
                    out_copy(i, so).wait()

                for r in range(_ROWS):
                    row = jnp.full((_LANES,), r, jnp.int32)

                    def inner(j, idx, row=row, si=si, so=so, r=r):
                        ev = plsc.load_gather(in_bufs[si], [row, idx])
                        out_bufs[so][r, pl.ds(j * _LANES, _LANES)] = ev
                        return idx + 32

                    lax.fori_loop(0, o_chunk // _LANES, inner, even,
                                  unroll=8)

                out_copy(i, so).start()
            return carry

        lax.fori_loop(0, chunks // _NIN, lambda g2, cr: outer(g2 * _NIN, cr),
                      0)
        for k in range(_NOUT):
            out_copy(chunks - _NOUT + k, (chunks - _NOUT + k) % _NOUT).wait()

    return resample


def kernel(x):
    b, c, t = x.shape
    fn = _make_resample(b, c, t)
    return fn(x)


# speedup vs baseline: 1.1205x; 1.1205x over previous
import functools

import jax
import jax.numpy as jnp
from jax import lax
from jax.experimental import pallas as pl
from jax.experimental.pallas import tpu as pltpu
from jax.experimental.pallas import tpu_sc as plsc

_LANES = 16
_NUM_WORKERS = 32
_ROWS = 8
_T_CHUNK = 4096


def _make_resample(b: int, c: int, t: int):
    o_chunk = _T_CHUNK // 2
    strips_total = (b * c) // _ROWS
    strips_per_w = strips_total // _NUM_WORKERS
    chunks_per_strip = t // _T_CHUNK
    chunks = strips_per_w * chunks_per_strip
    assert strips_per_w * _NUM_WORKERS == strips_total
    assert chunks_per_strip * _T_CHUNK == t and chunks % 2 == 0
    strips_per_batch = c // _ROWS

    mesh = plsc.VectorSubcoreMesh(core_axis_name="c", subcore_axis_name="s")

    @functools.partial(
        pl.kernel,
        mesh=mesh,
        compiler_params=pltpu.CompilerParams(needs_layout_passes=False),
        out_type=jax.ShapeDtypeStruct((b, c, t // 2), jnp.float32),
        scratch_types=[
            pltpu.VMEM((_ROWS, _T_CHUNK), jnp.float32),
            pltpu.VMEM((_ROWS, _T_CHUNK), jnp.float32),
            pltpu.VMEM((_ROWS, o_chunk), jnp.float32),
            pltpu.VMEM((_ROWS, o_chunk), jnp.float32),
            pltpu.SemaphoreType.DMA,
            pltpu.SemaphoreType.DMA,
            pltpu.SemaphoreType.DMA,
            pltpu.SemaphoreType.DMA,
        ],
    )
    def resample(x_hbm, out_hbm, in0, in1, out0, out1,
                 in_sem0, in_sem1, out_sem0, out_sem1):
        wid = lax.axis_index("s") * 2 + lax.axis_index("c")
        strip0 = wid * strips_per_w
        in_bufs = (in0, in1)
        out_bufs = (out0, out1)
        in_sems = (in_sem0, in_sem1)
        out_sems = (out_sem0, out_sem1)
        lane = lax.iota(jnp.int32, _LANES)
        dg_idx = (lane * 2) % _LANES
        lo_half = lane < 8
        _dnums = lax.GatherDimensionNumbers(
            offset_dims=(), collapsed_slice_dims=(0,), start_index_map=(0,))

        def _dg(v, idx):
            return lax.gather(
                v, idx[:, None], _dnums, slice_sizes=(1,),
                mode=lax.GatherScatterMode.PROMISE_IN_BOUNDS)

        def in_copy(i, bf):
            strip = strip0 + i // chunks_per_strip
            t0 = (i % chunks_per_strip) * _T_CHUNK
            src = x_hbm.at[strip // strips_per_batch,
                           pl.ds((strip % strips_per_batch) * _ROWS, _ROWS),
                           pl.ds(t0, _T_CHUNK)]
            return pltpu.make_async_copy(src, in_bufs[bf], in_sems[bf])

        def out_copy(i, bf):
            strip = strip0 + i // chunks_per_strip
            t0 = (i % chunks_per_strip) * o_chunk
            dst = out_hbm.at[strip // strips_per_batch,
                             pl.ds((strip % strips_per_batch) * _ROWS, _ROWS),
                             pl.ds(t0, o_chunk)]
            return pltpu.make_async_copy(out_bufs[bf], dst, out_sems[bf])

        in_copy(0, 0).start()
        in_copy(1, 1).start()

        def outer(g, carry):
            for bf in range(2):
                i = g + bf
                in_copy(i, bf).wait()

                @pl.when(i >= 2)
                def _():
                    out_copy(i, bf).wait()

                for r in range(_ROWS):
                    def inner(j, cr, bf=bf, r=r):
                        v1 = in_bufs[bf][r, pl.ds(j * 32, _LANES)]
                        v2 = in_bufs[bf][r, pl.ds(j * 32 + _LANES, _LANES)]
                        g1 = _dg(v1, dg_idx)
                        g2 = _dg(v2, dg_idx)
                        ev = jnp.where(lo_half, g1, g2)
                        out_bufs[bf][r, pl.ds(j * _LANES, _LANES)] = ev
                        return cr

                    lax.fori_loop(0, o_chunk // _LANES, inner, 0, unroll=4)

                out_copy(i, bf).start()

                @pl.when(i + 2 < chunks)
                def _():
                    in_copy(i + 2, bf).start()
            return carry

        lax.fori_loop(0, chunks // 2, lambda g2, cr: outer(g2 * 2, cr), 0)
        out_copy(chunks - 2, 0).wait()
        out_copy(chunks - 1, 1).wait()

    return resample


def kernel(x):
    b, c, t = x.shape
    fn = _make_resample(b, c, t)
    return fn(x)


# speedup vs baseline: 2.6825x; 2.3940x over previous
import functools

import jax
import jax.numpy as jnp
from jax import lax
from jax.experimental import pallas as pl
from jax.experimental.pallas import tpu as pltpu
from jax.experimental.pallas import tpu_sc as plsc

_LANES = 16
_NUM_WORKERS = 32
_ROWS = 8
_T_CHUNK = 4096


def _make_resample(b: int, c: int, t: int):
    o_chunk = _T_CHUNK // 2
    strips_total = (b * c) // _ROWS
    strips_per_w = strips_total // _NUM_WORKERS
    chunks_per_strip = t // _T_CHUNK
    chunks = strips_per_w * chunks_per_strip
    assert strips_per_w * _NUM_WORKERS == strips_total
    assert chunks_per_strip * _T_CHUNK == t and chunks % 2 == 0
    strips_per_batch = c // _ROWS

    mesh = plsc.VectorSubcoreMesh(core_axis_name="c", subcore_axis_name="s")

    @functools.partial(
        pl.kernel,
        mesh=mesh,
        compiler_params=pltpu.CompilerParams(needs_layout_passes=False),
        out_type=jax.ShapeDtypeStruct((b, c, t // 2), jnp.float32),
        scratch_types=[
            pltpu.VMEM((_ROWS, _T_CHUNK), jnp.float32),
            pltpu.VMEM((_ROWS, _T_CHUNK), jnp.float32),
            pltpu.VMEM((_ROWS, o_chunk), jnp.float32),
            pltpu.VMEM((_ROWS, o_chunk), jnp.float32),
            pltpu.SemaphoreType.DMA,
            pltpu.SemaphoreType.DMA,
            pltpu.SemaphoreType.DMA,
            pltpu.SemaphoreType.DMA,
        ],
    )
    def resample(x_hbm, out_hbm, in0, in1, out0, out1,
                 in_sem0, in_sem1, out_sem0, out_sem1):
        wid = lax.axis_index("s") * 2 + lax.axis_index("c")
        strip0 = wid * strips_per_w
        in_bufs = (in0, in1)
        out_bufs = (out0, out1)
        in_sems = (in_sem0, in_sem1)
        out_sems = (out_sem0, out_sem1)
        even = lax.iota(jnp.int32, _LANES) * 2

        def in_copy(i, bf):
            strip = strip0 + i // chunks_per_strip
            t0 = (i % chunks_per_strip) * _T_CHUNK
            src = x_hbm.at[strip // strips_per_batch,
                           pl.ds((strip % strips_per_batch) * _ROWS, _ROWS),
                           pl.ds(t0, _T_CHUNK)]
            return pltpu.make_async_copy(src, in_bufs[bf], in_sems[bf])

        def out_copy(i, bf):
            strip = strip0 + i // chunks_per_strip
            t0 = (i % chunks_per_strip) * o_chunk
            dst = out_hbm.at[strip // strips_per_batch,
                             pl.ds((strip % strips_per_batch) * _ROWS, _ROWS),
                             pl.ds(t0, o_chunk)]
            return pltpu.make_async_copy(out_bufs[bf], dst, out_sems[bf])

        in_copy(0, 0).start()
        in_copy(1, 1).start()

        def outer(g, carry):
            for bf in range(2):
                i = g + bf
                in_copy(i, bf).wait()

                @pl.when(i >= 2)
                def _():
                    out_copy(i, bf).wait()

                for r in range(_ROWS):
                    row = jnp.full((_LANES,), r, jnp.int32)

                    def inner(j, row=row, bf=bf, r=r):
                        ev = plsc.load_gather(
                            in_bufs[bf], [row, j * 32 + even])
                        out_bufs[bf][r, pl.ds(j * _LANES, _LANES)] = ev

                    plsc.parallel_loop(
                        0, o_chunk // _LANES, 1, unroll=8)(inner)

                out_copy(i, bf).start()

                @pl.when(i + 2 < chunks)
                def _():
                    in_copy(i + 2, bf).start()
            return carry

        lax.fori_loop(0, chunks // 2, lambda g2, cr: outer(g2 * 2, cr), 0)
        out_copy(chunks - 2, 0).wait()
        out_copy(chunks - 1, 1).wait()

    return resample


def kernel(x):
    b, c, t = x.shape
    fn = _make_resample(b, c, t)
    return fn(x)
